# baseline (speedup 1.0000x reference)
import functools

import jax
import jax.numpy as jnp
from jax import lax
from jax.experimental import pallas as pl
from jax.experimental.pallas import tpu as pltpu
from jax.experimental.pallas import tpu_sc as plsc

_B, _L, _D = 1024, 200, 64
_NC, _NS = 2, 16
_NW = _NC * _NS
_T = 4
_V = 100000
_NSEG = 8
_NFUSED = 16 * _NSEG
_LANES = 16

_NB = 8
_LG = _NW // _NB
_BPB = _B // _NB
_LPW = _L // _LG
_NBUF = 5
_G = _LPW // _NBUF


def _embed_body(cats_hbm, tt_hbm, seg_hbm, big_hbm, small_hbm, out_hbm,
                slab_v, fused_v, seg_v, acc_v, acct_v,
                sem_g0, sem_add, sem_wb):
    w = lax.axis_index("s") * _NC + lax.axis_index("c")
    l0 = lax.div(w, _NB) * _LPW
    jblk = lax.rem(w, _NB)
    b0 = jblk * _BPB

    pltpu.sync_copy(cats_hbm.at[pl.ds(l0, _LPW), :, pl.ds(b0, _BPB)], slab_v)
    pltpu.sync_copy(tt_hbm.at[pl.ds(l0, _LPW), pl.ds(b0, _BPB)], fused_v)
    pltpu.sync_copy(seg_hbm.at[pl.ds(l0, _LPW), pl.ds(b0, _BPB)], seg_v)

    def prep(li, carry):
        for g in range(_BPB // _LANES):
            s = pl.ds(g * _LANES, _LANES)
            for c in range(1, _T):
                slab_v[li, c, s] = slab_v[li, c, s] + c * _V
            fused_v[li, s] = (fused_v[li, s] * _NSEG + seg_v[li, s]
                              + (l0 + li) * _NFUSED)
        return carry

    lax.fori_loop(0, _LPW, prep, 0)

    def out_slice(li):
        return out_hbm.at[l0 + li, :, jblk, :, :]

    lane = lax.iota(jnp.int32, _LANES)

    def transpose_chunk(b):
        def col_body(d, carry):
            col = jnp.broadcast_to(d, (_LANES,))
            dt = lax.div(d, 8)
            ds = lax.rem(d, 8)
            for g in range(_BPB // _LANES):
                rows = g * _LANES + lane
                v = plsc.load_gather(acc_v.at[b], [rows, col])
                acct_v[b, dt, ds, pl.ds(g * _LANES, _LANES)] = v
            return carry

        lax.fori_loop(0, _D, col_body, 0)

    def fire_g0(li, b):
        pltpu.async_copy(big_hbm.at[slab_v.at[li, 0]], acc_v.at[b],
                         sem_g0.at[b])

    def fire_adds(li, b):
        pltpu.make_async_copy(
            big_hbm.at[slab_v.at[li, 0]], acc_v.at[b], sem_g0.at[b]).wait()
        for c in range(1, _T):
            pltpu.async_copy(big_hbm.at[slab_v.at[li, c]], acc_v.at[b],
                             sem_add.at[b], add=True)
        pltpu.async_copy(small_hbm.at[fused_v.at[li]], acc_v.at[b],
                         sem_add.at[b], add=True)

    def fire_wb(li, b):
        for _ in range(_T):
            pltpu.make_async_copy(
                big_hbm.at[slab_v.at[li, 1]], acc_v.at[b],
                sem_add.at[b]).wait()
        transpose_chunk(b)
        pltpu.async_copy(acct_v.at[b], out_slice(li), sem_wb.at[b])

    def wait_wb(li, b):
        pltpu.make_async_copy(acct_v.at[b], out_slice(li), sem_wb.at[b]).wait()

    for b in range(_NBUF):
        fire_g0(b, b)
    for b in range(_NBUF):
        fire_adds(b, b)

    def outer(g, carry):
        for b in range(_NBUF):
            fire_wb((g - 1) * _NBUF + b, b)
        for b in range(_NBUF):
            wait_wb((g - 1) * _NBUF + b, b)
            fire_g0(g * _NBUF + b, b)
        for b in range(_NBUF):
            fire_adds(g * _NBUF + b, b)
        return carry

    lax.fori_loop(1, _G, outer, 0)

    for b in range(_NBUF):
        fire_wb((_G - 1) * _NBUF + b, b)
    for b in range(_NBUF):
        wait_wb((_G - 1) * _NBUF + b, b)


_embed = functools.partial(
    pl.kernel,
    out_type=jax.ShapeDtypeStruct((_L, 8, _NB, _D // 8, _BPB), jnp.float32),
    mesh=plsc.VectorSubcoreMesh(core_axis_name="c", subcore_axis_name="s"),
    scratch_types=[
        pltpu.VMEM((_LPW, _T, _BPB), jnp.int32),
        pltpu.VMEM((_LPW, _BPB), jnp.int32),
        pltpu.VMEM((_LPW, _BPB), jnp.int32),
        pltpu.VMEM((_NBUF, _BPB, _D), jnp.float32),
        pltpu.VMEM((_NBUF, 8, _D // 8, _BPB), jnp.float32),
        pltpu.SemaphoreType.DMA((_NBUF,)),
        pltpu.SemaphoreType.DMA((_NBUF,)),
        pltpu.SemaphoreType.DMA((_NBUF,)),
    ],
    compiler_params=pltpu.CompilerParams(use_tc_tiling_on_sc=False,
                                         needs_layout_passes=False),
)(_embed_body)


def kernel(token_types, segments, semantic_embeds, categories,
           token_type_table, segment_table, cat_tables, pe):
    del semantic_embeds
    big = cat_tables.reshape(_T * _V, _D)
    fused_small = (token_type_table[:, None, :]
                   + segment_table[None, :, :]).reshape(-1, _D)
    small = (pe[0, :_L, None, :] + fused_small[None, :, :]).reshape(-1, _D)

    cats_lcb = jnp.transpose(categories.astype(jnp.int32), (1, 2, 0))
    tt_lb = token_types.astype(jnp.int32).T
    seg_lb = segments.astype(jnp.int32).T

    out = _embed(cats_lcb, tt_lb, seg_lb, big, small)
    return out.transpose(2, 4, 0, 1, 3).reshape(_B, _L, _D)


# speedup vs baseline: 1.3444x; 1.3444x over previous
import functools

import jax
import jax.numpy as jnp
from jax import lax
from jax.experimental import pallas as pl
from jax.experimental.pallas import tpu as pltpu
from jax.experimental.pallas import tpu_sc as plsc

_B, _L, _D = 1024, 200, 64
_NC, _NS = 2, 16
_NW = _NC * _NS
_T = 4
_V = 100000
_NSEG = 8
_NFUSED = 16 * _NSEG
_LANES = 16

_NB = 8
_LG = _NW // _NB
_BPB = _B // _NB
_LPW = _L // _LG
_NBUF = 5
_G = _LPW // _NBUF


def _embed_body(cats_hbm, tt_hbm, seg_hbm, big_hbm, small_hbm, out_hbm,
                slab_v, fused_v, seg_v, acc_v,
                sem_g0, sem_add, sem_wb):
    w = lax.axis_index("s") * _NC + lax.axis_index("c")
    l0 = lax.div(w, _NB) * _LPW
    jblk = lax.rem(w, _NB)
    b0 = jblk * _BPB

    pltpu.sync_copy(cats_hbm.at[pl.ds(l0, _LPW), :, pl.ds(b0, _BPB)], slab_v)
    pltpu.sync_copy(tt_hbm.at[pl.ds(l0, _LPW), pl.ds(b0, _BPB)], fused_v)
    pltpu.sync_copy(seg_hbm.at[pl.ds(l0, _LPW), pl.ds(b0, _BPB)], seg_v)

    def prep(li, carry):
        for g in range(_BPB // _LANES):
            s = pl.ds(g * _LANES, _LANES)
            for c in range(1, _T):
                slab_v[li, c, s] = slab_v[li, c, s] + c * _V
            fused_v[li, s] = (fused_v[li, s] * _NSEG + seg_v[li, s]
                              + (l0 + li) * _NFUSED)
        return carry

    lax.fori_loop(0, _LPW, prep, 0)

    def out_slice(li):
        return out_hbm.at[l0 + li, pl.ds(b0, _BPB), :]

    def fire_g0(li, b):
        pltpu.async_copy(big_hbm.at[slab_v.at[li, 0]], acc_v.at[b],
                         sem_g0.at[b])

    def fire_adds(li, b):
        pltpu.make_async_copy(
            big_hbm.at[slab_v.at[li, 0]], acc_v.at[b], sem_g0.at[b]).wait()
        for c in range(1, _T):
            pltpu.async_copy(big_hbm.at[slab_v.at[li, c]], acc_v.at[b],
                             sem_add.at[b], add=True)
        pltpu.async_copy(small_hbm.at[fused_v.at[li]], acc_v.at[b],
                         sem_add.at[b], add=True)

    def fire_wb(li, b):
        for _ in range(_T):
            pltpu.make_async_copy(
                big_hbm.at[slab_v.at[li, 1]], acc_v.at[b],
                sem_add.at[b]).wait()
        pltpu.async_copy(acc_v.at[b], out_slice(li), sem_wb.at[b])

    def wait_wb(li, b):
        pltpu.make_async_copy(acc_v.at[b], out_slice(li), sem_wb.at[b]).wait()

    for b in range(_NBUF):
        fire_g0(b, b)
    for b in range(_NBUF):
        fire_adds(b, b)

    def outer(g, carry):
        for b in range(_NBUF):
            fire_wb((g - 1) * _NBUF + b, b)
        for b in range(_NBUF):
            wait_wb((g - 1) * _NBUF + b, b)
            fire_g0(g * _NBUF + b, b)
        for b in range(_NBUF):
            fire_adds(g * _NBUF + b, b)
        return carry

    lax.fori_loop(1, _G, outer, 0)

    for b in range(_NBUF):
        fire_wb((_G - 1) * _NBUF + b, b)
    for b in range(_NBUF):
        wait_wb((_G - 1) * _NBUF + b, b)


_embed = functools.partial(
    pl.kernel,
    out_type=jax.ShapeDtypeStruct((_L, _B, _D), jnp.float32),
    mesh=plsc.VectorSubcoreMesh(core_axis_name="c", subcore_axis_name="s"),
    scratch_types=[
        pltpu.VMEM((_LPW, _T, _BPB), jnp.int32),
        pltpu.VMEM((_LPW, _BPB), jnp.int32),
        pltpu.VMEM((_LPW, _BPB), jnp.int32),
        pltpu.VMEM((_NBUF, _BPB, _D), jnp.float32),
        pltpu.SemaphoreType.DMA((_NBUF,)),
        pltpu.SemaphoreType.DMA((_NBUF,)),
        pltpu.SemaphoreType.DMA((_NBUF,)),
    ],
    compiler_params=pltpu.CompilerParams(use_tc_tiling_on_sc=False,
                                         needs_layout_passes=False),
)(_embed_body)


def kernel(token_types, segments, semantic_embeds, categories,
           token_type_table, segment_table, cat_tables, pe):
    del semantic_embeds
    big = cat_tables.reshape(_T * _V, _D)
    fused_small = (token_type_table[:, None, :]
                   + segment_table[None, :, :]).reshape(-1, _D)
    small = (pe[0, :_L, None, :] + fused_small[None, :, :]).reshape(-1, _D)

    cats_lcb = jnp.transpose(categories.astype(jnp.int32), (1, 2, 0))
    tt_lb = token_types.astype(jnp.int32).T
    seg_lb = segments.astype(jnp.int32).T

    out = _embed(cats_lcb, tt_lb, seg_lb, big, small)
    return out.transpose(1, 0, 2)
